# Initial kernel scaffold; baseline (speedup 1.0000x reference)
import jax
import jax.numpy as jnp
from jax.experimental import pallas as pl


def kernel(idx, targets, table):
    raise NotImplementedError("write your pallas kernel here")


# speedup vs baseline: 2.1505x; 2.1505x over previous
import functools

import jax
import jax.numpy as jnp
from jax import lax
from jax.experimental import pallas as pl
from jax.experimental.pallas import tpu as pltpu
from jax.experimental.pallas import tpu_sc as plsc

_VOCAB = 4096
_NW = 32
_CHUNK = 16
_ROWS_PER_W = 8192 // _NW
_CHUNKS_PER_W = _ROWS_PER_W // _CHUNK

_mesh = plsc.VectorSubcoreMesh(core_axis_name="c", subcore_axis_name="s")


@functools.partial(
    pl.kernel,
    mesh=_mesh,
    compiler_params=pltpu.CompilerParams(needs_layout_passes=False),
    out_type=[
        jax.ShapeDtypeStruct((8192, _VOCAB), jnp.float32),
        jax.ShapeDtypeStruct((_NW, _CHUNK), jnp.float32),
    ],
    scratch_types=[
        pltpu.VMEM((_CHUNKS_PER_W, _CHUNK), jnp.int32),
        pltpu.VMEM((_CHUNKS_PER_W, _CHUNK), jnp.int32),
        pltpu.VMEM((_VOCAB,), jnp.float32),
        pltpu.VMEM((_CHUNK, _VOCAB), jnp.float32),
        pltpu.VMEM((_CHUNK,), jnp.float32),
        pltpu.SemaphoreType.DMA,
    ],
)
def _sc_gather_loss(idx_hbm, tgt_hbm, lse_hbm, table_hbm, out_hbm, part_hbm,
                    idx_v, tgt_v, lse_v, rows_v, acc_v, sem):
    wid = lax.axis_index("s") * 2 + lax.axis_index("c")
    base = wid * _ROWS_PER_W
    pltpu.sync_copy(idx_hbm.at[wid], idx_v)
    pltpu.sync_copy(tgt_hbm.at[wid], tgt_v)
    pltpu.sync_copy(lse_hbm, lse_v)
    row_ids = lax.iota(jnp.int32, _CHUNK)

    def body(g, acc):
        pltpu.async_copy(table_hbm.at[idx_v.at[g]], rows_v, sem).wait()
        tvals = plsc.load_gather(rows_v, [row_ids, tgt_v[g]])
        lvals = plsc.load_gather(lse_v, [idx_v[g]])
        pltpu.sync_copy(rows_v, out_hbm.at[pl.ds(base + g * _CHUNK, _CHUNK)])
        return acc + lvals - tvals

    acc = lax.fori_loop(0, _CHUNKS_PER_W, body,
                        jnp.zeros((_CHUNK,), jnp.float32))
    acc_v[...] = acc
    pltpu.sync_copy(acc_v, part_hbm.at[wid])


def _lse_body(tab_ref, out_ref):
    x = tab_ref[...]
    m = jnp.max(x, axis=1, keepdims=True)
    out_ref[...] = jnp.log(jnp.sum(jnp.exp(x - m), axis=1)) + m[:, 0]


def _finalize_body(part_ref, out_ref):
    out_ref[...] = jnp.reshape(jnp.sum(part_ref[...]) * (1.0 / 8192.0), (1, 1))


def kernel(idx, targets, table):
    idx_c = idx.reshape(_NW, _CHUNKS_PER_W, _CHUNK)
    tgt_c = targets.reshape(_NW, _CHUNKS_PER_W, _CHUNK)

    lse = pl.pallas_call(
        _lse_body,
        grid=(16,),
        in_specs=[pl.BlockSpec((_VOCAB // 16, _VOCAB), lambda i: (i, 0))],
        out_specs=pl.BlockSpec((_VOCAB // 16,), lambda i: (i,)),
        out_shape=jax.ShapeDtypeStruct((_VOCAB,), jnp.float32),
    )(table)

    logits_flat, partials = _sc_gather_loss(idx_c, tgt_c, lse, table)

    loss = pl.pallas_call(
        _finalize_body,
        out_shape=jax.ShapeDtypeStruct((1, 1), jnp.float32),
    )(partials)[0, 0]

    return (logits_flat.reshape(idx.shape[0], idx.shape[1], _VOCAB), loss)


# speedup vs baseline: 2.3347x; 1.0856x over previous
import functools

import jax
import jax.numpy as jnp
from jax import lax
from jax.experimental import pallas as pl
from jax.experimental.pallas import tpu as pltpu
from jax.experimental.pallas import tpu_sc as plsc

_VOCAB = 4096
_NW = 32
_ROWS_PER_W = 8192 // _NW
_C = 8
_NCH = _ROWS_PER_W // _C
_L = 16

_mesh = plsc.VectorSubcoreMesh(core_axis_name="c", subcore_axis_name="s")
_sc_params = pltpu.CompilerParams(needs_layout_passes=False)


@functools.partial(
    pl.kernel,
    mesh=_mesh,
    compiler_params=_sc_params,
    out_type=[
        jax.ShapeDtypeStruct((8192, _VOCAB), jnp.float32),
        jax.ShapeDtypeStruct((_NW, _L), jnp.float32),
    ],
    scratch_types=[
        pltpu.VMEM((_NCH, _C), jnp.int32),
        pltpu.VMEM((_ROWS_PER_W + _L,), jnp.int32),
        pltpu.VMEM((_C, _VOCAB), jnp.float32),
        pltpu.VMEM((_C, _VOCAB), jnp.float32),
        pltpu.VMEM((_L,), jnp.float32),
        pltpu.SemaphoreType.DMA,
        pltpu.SemaphoreType.DMA,
        pltpu.SemaphoreType.DMA,
        pltpu.SemaphoreType.DMA,
    ],
)
def _sc_stream(idx_hbm, tgt_hbm, table_hbm, out_hbm, part_hbm,
               idx_v, tgt_v, rows_a, rows_b, acc_v,
               gsem_a, gsem_b, ssem_a, ssem_b):
    wid = lax.axis_index("s") * 2 + lax.axis_index("c")
    base = wid * _ROWS_PER_W
    pltpu.sync_copy(idx_hbm.at[wid], idx_v)
    pltpu.sync_copy(tgt_hbm.at[wid], tgt_v.at[pl.ds(0, _ROWS_PER_W)])
    tgt_v[pl.ds(_ROWS_PER_W, _L)] = jnp.zeros((_L,), jnp.int32)
    lanes = lax.iota(jnp.int32, _L)
    row_ids = lanes & (_C - 1)
    lo_half = lanes < _C

    def gather(g, buf, sem):
        pltpu.async_copy(table_hbm.at[idx_v.at[g]], buf, sem)

    def gather_wait(g, buf, sem):
        pltpu.make_async_copy(table_hbm.at[idx_v.at[g]], buf, sem).wait()

    def scatter(g, buf, sem):
        pltpu.async_copy(buf, out_hbm.at[pl.ds(base + g * _C, _C)], sem)

    def scatter_wait(g, buf, sem):
        pltpu.make_async_copy(
            buf, out_hbm.at[pl.ds(base + g * _C, _C)], sem).wait()

    def tval(buf, g):
        t16 = tgt_v[pl.ds(g * _C, _L)]
        v = plsc.load_gather(buf, [row_ids, t16])
        return jnp.where(lo_half, v, 0.0)

    gather(0, rows_a, gsem_a)

    def step(i, tacc):
        a = 2 * i
        b = a + 1
        gather_wait(a, rows_a, gsem_a)

        @pl.when(i > 0)
        def _():
            scatter_wait(b - 2, rows_b, ssem_b)

        gather(b, rows_b, gsem_b)
        scatter(a, rows_a, ssem_a)
        tacc = tacc + tval(rows_a, a)
        gather_wait(b, rows_b, gsem_b)
        scatter_wait(a, rows_a, ssem_a)

        @pl.when(i < _NCH // 2 - 1)
        def _():
            gather(a + 2, rows_a, gsem_a)

        scatter(b, rows_b, ssem_b)
        return tacc + tval(rows_b, b)

    tacc = lax.fori_loop(0, _NCH // 2, step, jnp.zeros((_L,), jnp.float32))
    scatter_wait(_NCH - 1, rows_b, ssem_b)
    acc_v[...] = tacc
    pltpu.sync_copy(acc_v, part_hbm.at[wid])


@functools.partial(
    pl.kernel,
    mesh=_mesh,
    compiler_params=_sc_params,
    out_type=jax.ShapeDtypeStruct((_NW, _L), jnp.float32),
    scratch_types=[
        pltpu.VMEM((_ROWS_PER_W // _L, _L), jnp.int32),
        pltpu.VMEM((_VOCAB,), jnp.float32),
        pltpu.VMEM((_L,), jnp.float32),
    ],
)
def _sc_lse_gather(idx_hbm, lse_hbm, part_hbm, idx_v, lse_v, acc_v):
    wid = lax.axis_index("s") * 2 + lax.axis_index("c")
    pltpu.sync_copy(idx_hbm.at[wid], idx_v)
    pltpu.sync_copy(lse_hbm, lse_v)

    def body(i, lacc):
        return lacc + plsc.load_gather(lse_v, [idx_v[i]])

    lacc = lax.fori_loop(0, _ROWS_PER_W // _L, body,
                         jnp.zeros((_L,), jnp.float32))
    acc_v[...] = lacc
    pltpu.sync_copy(acc_v, part_hbm.at[wid])


def _lse_body(tab_ref, out_ref):
    x = tab_ref[...]
    m = jnp.max(x, axis=1, keepdims=True)
    out_ref[...] = jnp.log(jnp.sum(jnp.exp(x - m), axis=1)) + m[:, 0]


def _finalize_body(lpart_ref, tpart_ref, out_ref):
    s = jnp.sum(lpart_ref[...]) - jnp.sum(tpart_ref[...])
    out_ref[...] = jnp.reshape(s * (1.0 / 8192.0), (1, 1))


def kernel(idx, targets, table):
    idx_c = idx.reshape(_NW, _NCH, _C)
    idx_l = idx.reshape(_NW, _ROWS_PER_W // _L, _L)
    tgt_c = targets.reshape(_NW, _ROWS_PER_W)

    logits_flat, tpart = _sc_stream(idx_c, tgt_c, table)

    lse = pl.pallas_call(
        _lse_body,
        grid=(16,),
        in_specs=[pl.BlockSpec((_VOCAB // 16, _VOCAB), lambda i: (i, 0))],
        out_specs=pl.BlockSpec((_VOCAB // 16,), lambda i: (i,)),
        out_shape=jax.ShapeDtypeStruct((_VOCAB,), jnp.float32),
    )(table)

    lpart = _sc_lse_gather(idx_l, lse)

    loss = pl.pallas_call(
        _finalize_body,
        out_shape=jax.ShapeDtypeStruct((1, 1), jnp.float32),
    )(lpart, tpart)[0, 0]

    return (logits_flat.reshape(idx.shape[0], idx.shape[1], _VOCAB), loss)


# speedup vs baseline: 2.3456x; 1.0047x over previous
import functools

import jax
import jax.numpy as jnp
from jax import lax
from jax.experimental import pallas as pl
from jax.experimental.pallas import tpu as pltpu
from jax.experimental.pallas import tpu_sc as plsc

_VOCAB = 4096
_NW = 32
_ROWS_PER_W = 8192 // _NW
_C = 16
_NCH = _ROWS_PER_W // _C
_L = 16
_VB = _VOCAB // 16

_mesh = plsc.VectorSubcoreMesh(core_axis_name="c", subcore_axis_name="s")
_sc_params = pltpu.CompilerParams(needs_layout_passes=False)


@functools.partial(
    pl.kernel,
    mesh=_mesh,
    compiler_params=_sc_params,
    out_type=[
        jax.ShapeDtypeStruct((8192, _VOCAB), jnp.float32),
        jax.ShapeDtypeStruct((_NW, _L), jnp.float32),
    ],
    scratch_types=[
        pltpu.VMEM((_NCH, _C), jnp.int32),
        pltpu.VMEM((_NCH, _C), jnp.int32),
        pltpu.VMEM((_C, _VOCAB), jnp.float32),
        pltpu.VMEM((_L,), jnp.float32),
        pltpu.SemaphoreType.DMA,
    ],
)
def _sc_stream(idx_hbm, tgt_hbm, table_hbm, out_hbm, part_hbm,
               idx_v, tgt_v, rows_v, acc_v, sem):
    wid = lax.axis_index("s") * 2 + lax.axis_index("c")
    base = wid * _ROWS_PER_W
    pltpu.sync_copy(idx_hbm.at[wid], idx_v)
    pltpu.sync_copy(tgt_hbm.at[wid], tgt_v)
    row_ids = lax.iota(jnp.int32, _L)

    def body(g, tacc):
        pltpu.async_copy(table_hbm.at[idx_v.at[g]], rows_v, sem).wait()
        tvals = plsc.load_gather(rows_v, [row_ids, tgt_v[g]])
        pltpu.sync_copy(rows_v, out_hbm.at[pl.ds(base + g * _C, _C)])
        return tacc + tvals

    tacc = lax.fori_loop(0, _NCH, body, jnp.zeros((_L,), jnp.float32))
    acc_v[...] = tacc
    pltpu.sync_copy(acc_v, part_hbm.at[wid])


def _lse_count_body(idx_ref, tab_ref, s1_ref):
    i = pl.program_id(0)
    x = tab_ref[...]
    m = jnp.max(x, axis=1, keepdims=True)
    lse = jnp.log(jnp.sum(jnp.exp(x - m), axis=1, keepdims=True)) + m
    rows = i * _VB + lax.broadcasted_iota(jnp.int32, (_VB, 1), 0)

    def cbody(j, cnt):
        ids = idx_ref[:, pl.ds(j * 1024, 1024)]
        eq = (ids == rows).astype(jnp.float32)
        return cnt + jnp.sum(eq, axis=1, keepdims=True)

    cnt = lax.fori_loop(0, 8, cbody, jnp.zeros((_VB, 1), jnp.float32))
    contrib = jnp.sum(cnt * lse).reshape(1, 1)

    @pl.when(i == 0)
    def _():
        s1_ref[...] = jnp.zeros((1, 1), jnp.float32)

    s1_ref[...] += contrib


def _finalize_body(s1_ref, tpart_ref, out_ref):
    s = s1_ref[0, 0] - jnp.sum(tpart_ref[...])
    out_ref[...] = jnp.reshape(s * (1.0 / 8192.0), (1, 1))


def kernel(idx, targets, table):
    idx_c = idx.reshape(_NW, _NCH, _C)
    tgt_c = targets.reshape(_NW, _NCH, _C)
    idx_row = idx.reshape(1, 8192)

    logits_flat, tpart = _sc_stream(idx_c, tgt_c, table)

    s1 = pl.pallas_call(
        _lse_count_body,
        grid=(16,),
        in_specs=[
            pl.BlockSpec((1, 8192), lambda i: (0, 0)),
            pl.BlockSpec((_VB, _VOCAB), lambda i: (i, 0)),
        ],
        out_specs=pl.BlockSpec((1, 1), lambda i: (0, 0)),
        out_shape=jax.ShapeDtypeStruct((1, 1), jnp.float32),
    )(idx_row, table)

    loss = pl.pallas_call(
        _finalize_body,
        out_shape=jax.ShapeDtypeStruct((1, 1), jnp.float32),
    )(s1, tpart)[0, 0]

    return (logits_flat.reshape(idx.shape[0], idx.shape[1], _VOCAB), loss)
